# baseline (speedup 1.0000x reference)
import jax
import jax.numpy as jnp
from jax.experimental import pallas as pl


def _copy_block(x_ref, o_ref):
    o_ref[...] = x_ref[...]


def kernel(edge_index):
    E = edge_index.shape[1]
    total = 2 * E
    C = 512
    R = total // C
    BR = 1000
    x = edge_index.reshape(R, C)
    out = pl.pallas_call(
        _copy_block,
        grid=(R // BR,),
        in_specs=[pl.BlockSpec((BR, C), lambda i: (i, 0))],
        out_specs=pl.BlockSpec((BR, C), lambda i: (i, 0)),
        out_shape=jax.ShapeDtypeStruct((R, C), edge_index.dtype),
    )(x)
    return out.reshape(2, E)
